# baseline (speedup 1.0000x reference)
import functools

import jax
import jax.numpy as jnp
from jax import lax
from jax.experimental import pallas as pl
from jax.experimental.pallas import tpu as pltpu
from jax.experimental.pallas import tpu_sc as plsc

D = 64
BLK = 128


def _build(batch: int, hist: int):
    info = plsc.get_sparse_core_info()
    nc, ns = info.num_cores, info.num_subcores
    nw = nc * ns
    assert batch == nw * BLK
    nsteps = hist
    ddim = D // 8
    bblocks = batch // BLK
    mesh = plsc.VectorSubcoreMesh(core_axis_name="c", subcore_axis_name="s")

    @functools.partial(
        pl.kernel,
        mesh=mesh,
        out_type=jax.ShapeDtypeStruct((hist, ddim, bblocks, 8, BLK), jnp.float32),
        scratch_types=[
            pltpu.VMEM((2, BLK), jnp.int32),
            pltpu.VMEM((2, BLK), jnp.int32),
            pltpu.VMEM((2, BLK, 128), jnp.float32),
            pltpu.VMEM((2, ddim, 8, BLK), jnp.float32),
            pltpu.SemaphoreType.DMA((2,)),
            pltpu.SemaphoreType.DMA((2,)),
            pltpu.SemaphoreType.DMA((2,)),
        ],
        compiler_params=pltpu.CompilerParams(needs_layout_passes=False),
    )
    def emb_kernel(ids_hbm, w_hbm, out_hbm, idx_v, pair_v, rows_v, rowsT, sem_i, sem_g, sem_o):
        bb = lax.axis_index("s") * nc + lax.axis_index("c")
        b0 = bb * BLK
        lanes = lax.iota(jnp.int32, 16)

        def wait_idx(b):
            pltpu.make_async_copy(
                ids_hbm.at[0, pl.ds(0, BLK)], idx_v.at[b], sem_i.at[b]
            ).wait()

        def wait_gather(b):
            pltpu.make_async_copy(
                w_hbm.at[pl.ds(0, BLK)], rows_v.at[b], sem_g.at[b]
            ).wait()

        def wait_out(b):
            pltpu.make_async_copy(
                rowsT.at[b], out_hbm.at[0, :, 0], sem_o.at[b]
            ).wait()

        pltpu.async_copy(ids_hbm.at[0, pl.ds(b0, BLK)], idx_v.at[0], sem_i.at[0])
        pltpu.async_copy(ids_hbm.at[1, pl.ds(b0, BLK)], idx_v.at[1], sem_i.at[1])

        def step(h, b):
            @pl.when(h < nsteps)
            def _():
                wait_idx(b)
                for grp in range(8):
                    v = idx_v[b, pl.ds(grp * 16, 16)]
                    pair_v[b, pl.ds(grp * 16, 16)] = lax.shift_right_logical(v, 1)

                @pl.when(h >= 2)
                def _():
                    wait_out(b)

                pltpu.async_copy(
                    w_hbm.at[pair_v.at[b]], rows_v.at[b], sem_g.at[b]
                )

            @pl.when(jnp.logical_and(h >= 1, h <= nsteps))
            def _():
                wait_gather(1 - b)
                for grp in range(8):
                    v = idx_v[1 - b, pl.ds(grp * 16, 16)]
                    half = lax.mul(lax.bitwise_and(v, 1), 64)
                    rvec = lanes + grp * 16
                    for dd in range(ddim):
                        for ds in range(8):
                            cvec = half + (dd * 8 + ds)
                            g = plsc.load_gather(rows_v.at[1 - b], [rvec, cvec])
                            rowsT[1 - b, dd, ds, pl.ds(grp * 16, 16)] = g
                pltpu.async_copy(
                    rowsT.at[1 - b], out_hbm.at[h - 1, :, bb], sem_o.at[1 - b]
                )

                @pl.when(h + 1 < nsteps)
                def _():
                    pltpu.async_copy(
                        ids_hbm.at[h + 1, pl.ds(b0, BLK)],
                        idx_v.at[1 - b],
                        sem_i.at[1 - b],
                    )

        def pairs(p, carry):
            step(2 * p, 0)
            step(2 * p + 1, 1)
            return carry

        lax.fori_loop(0, (nsteps + 2) // 2, pairs, 0)

        wait_out(nsteps % 2)
        wait_out(1 - nsteps % 2)

    return emb_kernel


def kernel(input_ids, weight):
    batch, hist = input_ids.shape
    ids_t = jnp.transpose(input_ids).astype(jnp.int32)
    w_pairs = jnp.reshape(weight, (weight.shape[0] // 2, 128))
    out5 = _build(batch, hist)(ids_t, w_pairs)
    out = jnp.transpose(out5, (2, 4, 0, 1, 3)).reshape(batch, hist, D)
    return out


# speedup vs baseline: 1.4167x; 1.4167x over previous
import functools

import jax
import jax.numpy as jnp
from jax import lax
from jax.experimental import pallas as pl
from jax.experimental.pallas import tpu as pltpu
from jax.experimental.pallas import tpu_sc as plsc

D = 64
BLK = 128


def _build(batch: int, hist: int):
    info = plsc.get_sparse_core_info()
    nc, ns = info.num_cores, info.num_subcores
    nw = nc * ns
    assert batch == nw * BLK
    nsteps = hist
    ddim = D // 8
    bblocks = batch // BLK
    mesh = plsc.VectorSubcoreMesh(core_axis_name="c", subcore_axis_name="s")

    @functools.partial(
        pl.kernel,
        mesh=mesh,
        out_type=jax.ShapeDtypeStruct((hist, ddim, bblocks, 8, BLK), jnp.float32),
        scratch_types=[
            pltpu.VMEM((2, BLK), jnp.int32),
            pltpu.VMEM((2, BLK), jnp.int32),
            pltpu.VMEM((2, BLK, 128), jnp.float32),
            pltpu.VMEM((2, ddim, 8, BLK), jnp.float32),
            pltpu.SemaphoreType.DMA((2,)),
            pltpu.SemaphoreType.DMA((2,)),
            pltpu.SemaphoreType.DMA((2,)),
        ],
        compiler_params=pltpu.CompilerParams(needs_layout_passes=False),
    )
    def emb_kernel(ids_hbm, w_hbm, out_hbm, idx_v, pair_v, rows_v, rowsT, sem_i, sem_g, sem_o):
        bb = lax.axis_index("s") * nc + lax.axis_index("c")
        b0 = bb * BLK
        lanes = lax.iota(jnp.int32, 16)

        def wait_idx(b):
            pltpu.make_async_copy(
                ids_hbm.at[0, pl.ds(0, BLK)], idx_v.at[b], sem_i.at[b]
            ).wait()

        def wait_gather(b):
            pltpu.make_async_copy(
                w_hbm.at[pl.ds(0, BLK)], rows_v.at[b], sem_g.at[b]
            ).wait()

        def wait_out(b):
            pltpu.make_async_copy(
                rowsT.at[b], out_hbm.at[0, :, 0], sem_o.at[b]
            ).wait()

        pltpu.async_copy(ids_hbm.at[0, pl.ds(b0, BLK)], idx_v.at[0], sem_i.at[0])
        pltpu.async_copy(ids_hbm.at[1, pl.ds(b0, BLK)], idx_v.at[1], sem_i.at[1])

        def step(h, b):
            @pl.when(h < nsteps)
            def _():
                wait_idx(b)
                for grp in range(8):
                    v = idx_v[b, pl.ds(grp * 16, 16)]
                    pair_v[b, pl.ds(grp * 16, 16)] = lax.shift_right_logical(v, 1)

                @pl.when(h >= 2)
                def _():
                    wait_out(b)

                pltpu.async_copy(
                    w_hbm.at[pair_v.at[b]], rows_v.at[b], sem_g.at[b]
                )

            @pl.when(jnp.logical_and(h >= 1, h <= nsteps))
            def _():
                wait_gather(1 - b)
                for grp in range(8):
                    v = idx_v[1 - b, pl.ds(grp * 16, 16)]
                    half = lax.mul(lax.bitwise_and(v, 1), 64)
                    rvec = lanes + grp * 16

                    @plsc.parallel_loop(0, D, 1, unroll=8)
                    def _dloop(d):
                        cvec = half + d
                        g = plsc.load_gather(rows_v.at[1 - b], [rvec, cvec])
                        rowsT[
                            1 - b,
                            lax.shift_right_logical(d, 3),
                            lax.bitwise_and(d, 7),
                            pl.ds(grp * 16, 16),
                        ] = g
                pltpu.async_copy(
                    rowsT.at[1 - b], out_hbm.at[h - 1, :, bb], sem_o.at[1 - b]
                )

                @pl.when(h + 1 < nsteps)
                def _():
                    pltpu.async_copy(
                        ids_hbm.at[h + 1, pl.ds(b0, BLK)],
                        idx_v.at[1 - b],
                        sem_i.at[1 - b],
                    )

        def pairs(p, carry):
            step(2 * p, 0)
            step(2 * p + 1, 1)
            return carry

        lax.fori_loop(0, (nsteps + 2) // 2, pairs, 0)

        wait_out(nsteps % 2)
        wait_out(1 - nsteps % 2)

    return emb_kernel


def kernel(input_ids, weight):
    batch, hist = input_ids.shape
    ids_t = jnp.transpose(input_ids).astype(jnp.int32)
    w_pairs = jnp.reshape(weight, (weight.shape[0] // 2, 128))
    out5 = _build(batch, hist)(ids_t, w_pairs)
    out = jnp.transpose(out5, (2, 4, 0, 1, 3)).reshape(batch, hist, D)
    return out


# speedup vs baseline: 1.5218x; 1.0742x over previous
import functools

import jax
import jax.numpy as jnp
from jax import lax
from jax.experimental import pallas as pl
from jax.experimental.pallas import tpu as pltpu
from jax.experimental.pallas import tpu_sc as plsc

D = 64
BLK = 128
NS = 4
LAG = 3


def _build(batch: int, hist: int):
    info = plsc.get_sparse_core_info()
    nc, ns = info.num_cores, info.num_subcores
    nw = nc * ns
    assert batch == nw * BLK
    nsteps = hist
    ddim = D // 8
    bblocks = batch // BLK
    mesh = plsc.VectorSubcoreMesh(core_axis_name="c", subcore_axis_name="s")

    @functools.partial(
        pl.kernel,
        mesh=mesh,
        out_type=jax.ShapeDtypeStruct((hist, ddim, bblocks, 8, BLK), jnp.float32),
        scratch_types=[
            pltpu.VMEM((NS, BLK), jnp.int32),
            pltpu.VMEM((NS, BLK), jnp.int32),
            pltpu.VMEM((NS, BLK), jnp.int32),
            pltpu.VMEM((NS, BLK, 128), jnp.float32),
            pltpu.VMEM((NS, ddim, 8, BLK), jnp.float32),
            pltpu.SemaphoreType.DMA((NS,)),
            pltpu.SemaphoreType.DMA((NS,)),
            pltpu.SemaphoreType.DMA((NS,)),
        ],
        compiler_params=pltpu.CompilerParams(needs_layout_passes=False),
    )
    def emb_kernel(
        ids_hbm, w_hbm, out_hbm, idx_v, pair_v, half_v, rows_v, rowsT,
        sem_i, sem_g, sem_o,
    ):
        bb = lax.axis_index("s") * nc + lax.axis_index("c")
        b0 = bb * BLK
        lanes = lax.iota(jnp.int32, 16)

        def wait_idx(b):
            pltpu.make_async_copy(
                ids_hbm.at[0, pl.ds(0, BLK)], idx_v.at[b], sem_i.at[b]
            ).wait()

        def wait_gather(b):
            pltpu.make_async_copy(
                w_hbm.at[pl.ds(0, BLK)], rows_v.at[b], sem_g.at[b]
            ).wait()

        def wait_out(b):
            pltpu.make_async_copy(
                rowsT.at[b], out_hbm.at[0, :, 0], sem_o.at[b]
            ).wait()

        for j in range(NS):
            pltpu.async_copy(
                ids_hbm.at[j, pl.ds(b0, BLK)], idx_v.at[j], sem_i.at[j]
            )

        def step(h, b):
            @pl.when(h < nsteps)
            def _():
                wait_idx(b)
                for grp in range(8):
                    v = idx_v[b, pl.ds(grp * 16, 16)]
                    pair_v[b, pl.ds(grp * 16, 16)] = lax.shift_right_logical(v, 1)
                    half_v[b, pl.ds(grp * 16, 16)] = lax.mul(
                        lax.bitwise_and(v, 1), 64
                    )

                @pl.when(h + NS < nsteps)
                def _():
                    pltpu.async_copy(
                        ids_hbm.at[h + NS, pl.ds(b0, BLK)],
                        idx_v.at[b],
                        sem_i.at[b],
                    )

                pltpu.async_copy(
                    w_hbm.at[pair_v.at[b]], rows_v.at[b], sem_g.at[b]
                )

            @pl.when(jnp.logical_and(h >= LAG, h < nsteps + LAG))
            def _():
                g = h - LAG
                bg = (b + NS - LAG) % NS

                @pl.when(h >= NS + LAG)
                def _():
                    wait_out(bg)

                wait_gather(bg)
                for grp in range(8):
                    half = half_v[bg, pl.ds(grp * 16, 16)]
                    rvec = lanes + grp * 16

                    @plsc.parallel_loop(0, D, 1, unroll=8)
                    def _dloop(d):
                        cvec = half + d
                        gat = plsc.load_gather(rows_v.at[bg], [rvec, cvec])
                        rowsT[
                            bg,
                            lax.shift_right_logical(d, 3),
                            lax.bitwise_and(d, 7),
                            pl.ds(grp * 16, 16),
                        ] = gat

                pltpu.async_copy(
                    rowsT.at[bg], out_hbm.at[g, :, bb], sem_o.at[bg]
                )

        def quad(p, carry):
            for q in range(NS):
                step(NS * p + q, q)
            return carry

        total = nsteps + LAG
        lax.fori_loop(0, (total + NS - 1) // NS, quad, 0)

        for j in range(NS):
            wait_out(j)

    return emb_kernel


def kernel(input_ids, weight):
    batch, hist = input_ids.shape
    ids_t = jnp.transpose(input_ids).astype(jnp.int32)
    w_pairs = jnp.reshape(weight, (weight.shape[0] // 2, 128))
    out5 = _build(batch, hist)(ids_t, w_pairs)
    out = jnp.transpose(out5, (2, 4, 0, 1, 3)).reshape(batch, hist, D)
    return out


# speedup vs baseline: 3.9519x; 2.5969x over previous
import functools

import jax
import jax.numpy as jnp
from jax import lax
from jax.experimental import pallas as pl
from jax.experimental.pallas import tpu as pltpu
from jax.experimental.pallas import tpu_sc as plsc

D = 64
BLK = 128
NS = 4
LAG = 3
W1 = 256

_mesh = lambda: plsc.VectorSubcoreMesh(core_axis_name="c", subcore_axis_name="s")
_params = pltpu.CompilerParams(needs_layout_passes=False)


def _build_pass1(nvocab: int):
    full_cols = (nvocab // W1) * W1
    nchunks = full_cols // W1
    info = plsc.get_sparse_core_info()
    nc, nw = info.num_cores, info.num_cores * info.num_subcores
    per_w = (nchunks + nw - 1) // nw
    tail_rows = (nvocab - full_cols) // 2

    @functools.partial(
        pl.kernel,
        mesh=_mesh(),
        out_type=jax.ShapeDtypeStruct((nvocab // 2, 128), jnp.float32),
        scratch_types=[
            pltpu.VMEM((2, D, W1), jnp.float32),
            pltpu.VMEM(((W1 // 2) * 134,), jnp.float32),
            pltpu.VMEM((2, W1 // 2, 128), jnp.float32),
            pltpu.VMEM((tail_rows, 128), jnp.float32),
            pltpu.SemaphoreType.DMA((2,)),
            pltpu.SemaphoreType.DMA((2,)),
        ],
        compiler_params=_params,
    )
    def pass1(wt_hbm, tail_hbm, out_hbm, slab, b134, bpack, tailb, sem_s, sem_o):
        wid = lax.axis_index("s") * nc + lax.axis_index("c")
        lanes = lax.iota(jnp.int32, 16)
        cvec = (
            lax.mul(lax.shift_right_logical(lanes, 1), 134)
            + lax.mul(lax.bitwise_and(lanes, 1), 65)
        )

        def chunk_id(k):
            return wid + k * nw

        def fire_slab(k, b):
            @pl.when(chunk_id(k) < nchunks)
            def _():
                pltpu.async_copy(
                    wt_hbm.at[:, pl.ds(chunk_id(k) * W1, W1)],
                    slab.at[b],
                    sem_s.at[b],
                )

        def wait_slab(b):
            pltpu.make_async_copy(
                wt_hbm.at[:, pl.ds(0, W1)], slab.at[b], sem_s.at[b]
            ).wait()

        def wait_out(b):
            pltpu.make_async_copy(
                bpack.at[b], out_hbm.at[pl.ds(0, W1 // 2)], sem_o.at[b]
            ).wait()

        @pl.when(wid == 0)
        def _():
            pltpu.sync_copy(tail_hbm, tailb)
            pltpu.sync_copy(tailb, out_hbm.at[pl.ds(full_cols // 2, tail_rows)])

        fire_slab(0, 0)

        def body(k, b):
            @pl.when(chunk_id(k) < nchunks)
            def _():
                wait_slab(b)
                fire_slab(k + 1, 1 - b)

                @plsc.parallel_loop(0, D * (W1 // 16), 1, unroll=8)
                def _scat(kk):
                    d = lax.shift_right_logical(kk, 4)
                    jb = lax.bitwise_and(kk, 15)
                    vec = slab[b, d, pl.ds(jb * 16, 16)]
                    idx = cvec + (jb * (8 * 134) + d)
                    plsc.store_scatter(b134, [idx], vec)

                @plsc.parallel_loop(0, (W1 // 2) * 8, 1, unroll=8)
                def _pack(kk):
                    p = lax.shift_right_logical(kk, 3)
                    cb = lax.bitwise_and(kk, 7)
                    off = p * 134 + jnp.where(cb < 4, cb * 16, 65 + (cb - 4) * 16)
                    v = b134[pl.ds(off, 16)]
                    bpack[b, p, pl.ds(cb * 16, 16)] = v

                @pl.when(k >= 2)
                def _():
                    wait_out(b)

                pltpu.async_copy(
                    bpack.at[b],
                    out_hbm.at[pl.ds(chunk_id(k) * (W1 // 2), W1 // 2)],
                    sem_o.at[b],
                )

        def pair(p, carry):
            body(2 * p, 0)
            body(2 * p + 1, 1)
            return carry

        lax.fori_loop(0, (per_w + 1) // 2, pair, 0)
        wait_out(0)
        wait_out(1)

    return pass1


def _build_pass2(batch: int, hist: int):
    info = plsc.get_sparse_core_info()
    nc, ns = info.num_cores, info.num_subcores
    nw = nc * ns
    assert batch == nw * BLK
    nsteps = hist
    ddim = D // 8
    bblocks = batch // BLK

    @functools.partial(
        pl.kernel,
        mesh=_mesh(),
        out_type=jax.ShapeDtypeStruct((hist, ddim, bblocks, 8, BLK), jnp.float32),
        scratch_types=[
            pltpu.VMEM((NS, BLK), jnp.int32),
            pltpu.VMEM((NS, BLK), jnp.int32),
            pltpu.VMEM((NS, BLK + 16), jnp.int32),
            pltpu.VMEM((NS, BLK, 128), jnp.float32),
            pltpu.VMEM((D * 135,), jnp.float32),
            pltpu.VMEM((NS, ddim, 8, BLK), jnp.float32),
            pltpu.SemaphoreType.DMA((NS,)),
            pltpu.SemaphoreType.DMA((NS,)),
            pltpu.SemaphoreType.DMA((NS,)),
        ],
        compiler_params=_params,
    )
    def pass2(
        ids_hbm, w_hbm, out_hbm, idx_v, pair_v, half_v, rows_v, t135, rowsP,
        sem_i, sem_g, sem_o,
    ):
        bb = lax.axis_index("s") * nc + lax.axis_index("c")
        b0 = bb * BLK
        lanes = lax.iota(jnp.int32, 16)
        lanes135 = lanes * 135

        def wait_idx(b):
            pltpu.make_async_copy(
                ids_hbm.at[0, pl.ds(0, BLK)], idx_v.at[b], sem_i.at[b]
            ).wait()

        def wait_gather(b):
            pltpu.make_async_copy(
                w_hbm.at[pl.ds(0, BLK)], rows_v.at[b], sem_g.at[b]
            ).wait()

        def wait_out(b):
            pltpu.make_async_copy(
                rowsP.at[b], out_hbm.at[0, :, 0], sem_o.at[b]
            ).wait()

        for j in range(NS):
            pltpu.async_copy(
                ids_hbm.at[j, pl.ds(b0, BLK)], idx_v.at[j], sem_i.at[j]
            )

        def step(h, b):
            @pl.when(h < nsteps)
            def _():
                wait_idx(b)
                for grp in range(8):
                    v = idx_v[b, pl.ds(grp * 16, 16)]
                    pair_v[b, pl.ds(grp * 16, 16)] = lax.shift_right_logical(v, 1)
                    half_v[b, pl.ds(grp * 16, 16)] = lax.mul(
                        lax.bitwise_and(v, 1), 64
                    )

                @pl.when(h + NS < nsteps)
                def _():
                    pltpu.async_copy(
                        ids_hbm.at[h + NS, pl.ds(b0, BLK)],
                        idx_v.at[b],
                        sem_i.at[b],
                    )

                pltpu.async_copy(
                    w_hbm.at[pair_v.at[b]], rows_v.at[b], sem_g.at[b]
                )

            @pl.when(jnp.logical_and(h >= LAG, h < nsteps + LAG))
            def _():
                g = h - LAG
                bg = (b + NS - LAG) % NS

                @pl.when(h >= NS + LAG)
                def _():
                    wait_out(bg)

                wait_gather(bg)

                @plsc.parallel_loop(0, BLK, 1, unroll=4)
                def _scat(i):
                    half = half_v[bg, pl.ds(i, 16)][0]
                    for dblk in range(4):
                        vec = rows_v[bg, i, pl.ds(half + dblk * 16, 16)]
                        idx = lanes135 + (dblk * (16 * 135) + i)
                        plsc.store_scatter(t135, [idx], vec)

                @plsc.parallel_loop(0, D * 8, 1, unroll=8)
                def _pack(kk):
                    r = lax.shift_right_logical(kk, 3)
                    cb = lax.bitwise_and(kk, 7)
                    v = t135[pl.ds(r * 135 + cb * 16, 16)]
                    rowsP[
                        bg,
                        lax.shift_right_logical(r, 3),
                        lax.bitwise_and(r, 7),
                        pl.ds(cb * 16, 16),
                    ] = v

                pltpu.async_copy(
                    rowsP.at[bg], out_hbm.at[g, :, bb], sem_o.at[bg]
                )

        def quad(p, carry):
            for q in range(NS):
                step(NS * p + q, q)
            return carry

        total = nsteps + LAG
        lax.fori_loop(0, (total + NS - 1) // NS, quad, 0)
        for j in range(NS):
            wait_out(j)

    return pass2


def kernel(input_ids, weight):
    batch, hist = input_ids.shape
    nvocab = weight.shape[0]
    ids_t = jnp.transpose(input_ids).astype(jnp.int32)
    w_t = jnp.transpose(weight)
    full = (nvocab // W1) * W1
    w_tail = weight[full:].reshape((nvocab - full) // 2, 128)
    w_pairs = _build_pass1(nvocab)(w_t, w_tail)
    out5 = _build_pass2(batch, hist)(ids_t, w_pairs)
    out = jnp.transpose(out5, (2, 4, 0, 1, 3)).reshape(batch, hist, D)
    return out
